# Initial kernel scaffold; baseline (speedup 1.0000x reference)
import jax
import jax.numpy as jnp
from jax.experimental import pallas as pl


def kernel(inputs, table):
    raise NotImplementedError("write your pallas kernel here")


# speedup vs baseline: 1.2785x; 1.2785x over previous
import functools

import jax
import jax.numpy as jnp
from jax import lax
from jax.experimental import pallas as pl
from jax.experimental.pallas import tpu as pltpu
from jax.experimental.pallas import tpu_sc as plsc

EMBED_DIM = 128
NUM_CORES = 2
NUM_SUBCORES = 16
NUM_WORKERS = NUM_CORES * NUM_SUBCORES
CHUNK = 128


def _make_emb_kernel(total_indices: int):
  per_worker = total_indices // NUM_WORKERS
  n_chunks = per_worker // CHUNK
  mesh = plsc.VectorSubcoreMesh(
      core_axis_name="c", subcore_axis_name="s",
      num_cores=NUM_CORES, num_subcores=NUM_SUBCORES)

  @functools.partial(
      pl.kernel,
      out_type=jax.ShapeDtypeStruct((total_indices, EMBED_DIM), jnp.float32),
      mesh=mesh,
      scratch_types=[
          pltpu.VMEM((n_chunks, CHUNK), jnp.int32),
          pltpu.VMEM((CHUNK, EMBED_DIM), jnp.float32),
          pltpu.SemaphoreType.DMA,
      ],
  )
  def emb_kernel(table_hbm, idx_hbm, out_hbm, idx_v, rows_v, sem):
    wid = lax.axis_index("s") * NUM_CORES + lax.axis_index("c")
    base = wid * per_worker
    pltpu.sync_copy(idx_hbm.at[wid], idx_v)

    def body(j, carry):
      pltpu.async_copy(table_hbm.at[idx_v.at[j]], rows_v, sem).wait()
      pltpu.sync_copy(rows_v, out_hbm.at[pl.ds(base + j * CHUNK, CHUNK)])
      return carry

    lax.fori_loop(0, n_chunks, body, 0)

  return emb_kernel


def kernel(inputs, table):
  batch, hist = inputs.shape
  total = batch * hist
  idx = inputs.astype(jnp.int32).reshape(
      NUM_WORKERS, total // (NUM_WORKERS * CHUNK), CHUNK)
  out = _make_emb_kernel(total)(table, idx)
  return out.reshape(batch, hist, EMBED_DIM)


# speedup vs baseline: 1.8617x; 1.4561x over previous
import functools

import jax
import jax.numpy as jnp
from jax import lax
from jax.experimental import pallas as pl
from jax.experimental.pallas import tpu as pltpu
from jax.experimental.pallas import tpu_sc as plsc

EMBED_DIM = 128
NUM_CORES = 2
NUM_SUBCORES = 16
NUM_WORKERS = NUM_CORES * NUM_SUBCORES
CHUNK = 128


def _make_emb_kernel(total_indices: int):
  per_worker = total_indices // NUM_WORKERS
  n_chunks = per_worker // CHUNK
  mesh = plsc.VectorSubcoreMesh(
      core_axis_name="c", subcore_axis_name="s",
      num_cores=NUM_CORES, num_subcores=NUM_SUBCORES)

  assert n_chunks % 2 == 0 and n_chunks >= 4

  @functools.partial(
      pl.kernel,
      out_type=jax.ShapeDtypeStruct((total_indices, EMBED_DIM), jnp.float32),
      mesh=mesh,
      scratch_types=[
          pltpu.VMEM((n_chunks, CHUNK), jnp.int32),
          pltpu.VMEM((CHUNK, EMBED_DIM), jnp.float32),
          pltpu.VMEM((CHUNK, EMBED_DIM), jnp.float32),
          pltpu.SemaphoreType.DMA,
          pltpu.SemaphoreType.DMA,
      ],
  )
  def emb_kernel(table_hbm, idx_hbm, out_hbm, idx_v, rows_a, rows_b, sem_a,
                 sem_b):
    wid = lax.axis_index("s") * NUM_CORES + lax.axis_index("c")
    base = wid * per_worker
    pltpu.sync_copy(idx_hbm.at[wid], idx_v)

    def gather(j, buf, sem):
      pltpu.async_copy(table_hbm.at[idx_v.at[j]], buf, sem)

    def gather_wait(j, buf, sem):
      pltpu.make_async_copy(table_hbm.at[idx_v.at[j]], buf, sem).wait()

    def write_out(j, buf):
      pltpu.sync_copy(buf, out_hbm.at[pl.ds(base + j * CHUNK, CHUNK)])

    gather(0, rows_a, sem_a)
    gather(1, rows_b, sem_b)

    def body(i, carry):
      j = 2 * i
      gather_wait(j, rows_a, sem_a)
      write_out(j, rows_a)
      gather(j + 2, rows_a, sem_a)
      gather_wait(j + 1, rows_b, sem_b)
      write_out(j + 1, rows_b)
      gather(j + 3, rows_b, sem_b)
      return carry

    lax.fori_loop(0, n_chunks // 2 - 1, body, 0)
    j = n_chunks - 2
    gather_wait(j, rows_a, sem_a)
    write_out(j, rows_a)
    gather_wait(j + 1, rows_b, sem_b)
    write_out(j + 1, rows_b)

  return emb_kernel


def kernel(inputs, table):
  batch, hist = inputs.shape
  total = batch * hist
  idx = inputs.astype(jnp.int32).reshape(
      NUM_WORKERS, total // (NUM_WORKERS * CHUNK), CHUNK)
  out = _make_emb_kernel(total)(table, idx)
  return out.reshape(batch, hist, EMBED_DIM)
